# Initial kernel scaffold; baseline (speedup 1.0000x reference)
import functools

import jax
import jax.numpy as jnp
from jax import lax
from jax.experimental import pallas as pl
from jax.experimental.pallas import tpu as pltpu
from jax.experimental.pallas import tpu_sc as plsc

N = 10000
D = 128
E = 320000

NPAD = 10240
NC, NS = 2, 16
NW = NC * NS
CHUNK = 128
CPW = 80
EPW = CPW * CHUNK
EPAD = NW * EPW
GROUP = 4
NGROUP = CPW // GROUP
RPT = NPAD // NS

_mesh = plsc.VectorSubcoreMesh(core_axis_name="c", subcore_axis_name="s")


@functools.partial(
    pl.kernel,
    out_type=(
        jax.ShapeDtypeStruct((NPAD, D), jnp.float32),
        jax.ShapeDtypeStruct((NPAD, D), jnp.float32),
    ),
    mesh=_mesh,
    scratch_types=[
        pltpu.VMEM((CPW, CHUNK), jnp.int32),
        pltpu.VMEM((CPW, CHUNK), jnp.int32),
        pltpu.VMEM((GROUP * CHUNK, D), jnp.float32),
        pltpu.VMEM_SHARED((NPAD, D), jnp.float32),
        pltpu.SemaphoreType.DMA,
        pltpu.SemaphoreType.DMA,
    ],
)
def _sc_agg(g_hbm, src_hbm, dst_hbm, out0, out1,
            srcbuf, dstbuf, rows, acc_sh, gsem, ssem):
    c = lax.axis_index("c")
    s = lax.axis_index("s")
    wid = s * NC + c

    def _zrow(i, carry):
        for j in range(D // 16):
            rows[i, pl.ds(j * 16, 16)] = jnp.zeros((16,), jnp.float32)
        return carry
    lax.fori_loop(0, GROUP * CHUNK, _zrow, 0)
    base = s * RPT
    pltpu.sync_copy(rows, acc_sh.at[pl.ds(base, GROUP * CHUNK)])
    pltpu.sync_copy(rows.at[pl.ds(0, RPT - GROUP * CHUNK)],
                    acc_sh.at[pl.ds(base + GROUP * CHUNK, RPT - GROUP * CHUNK)])

    pltpu.sync_copy(src_hbm.at[wid], srcbuf)
    pltpu.sync_copy(dst_hbm.at[wid], dstbuf)
    plsc.subcore_barrier()

    def _group(gi, carry):
        gathers = []
        for b in range(GROUP):
            gathers.append(pltpu.async_copy(
                g_hbm.at[srcbuf.at[gi * GROUP + b]],
                rows.at[pl.ds(b * CHUNK, CHUNK)], gsem))
        scatters = []
        for b in range(GROUP):
            gathers[b].wait()
            scatters.append(pltpu.async_copy(
                rows.at[pl.ds(b * CHUNK, CHUNK)],
                acc_sh.at[dstbuf.at[gi * GROUP + b]], ssem, add=True))
        for b in range(GROUP):
            scatters[b].wait()
        return carry
    lax.fori_loop(0, NGROUP, _group, 0)
    plsc.subcore_barrier()

    @pl.when(c == 0)
    def _():
        pltpu.sync_copy(acc_sh.at[pl.ds(base, RPT)], out0.at[pl.ds(base, RPT)])

    @pl.when(c == 1)
    def _():
        pltpu.sync_copy(acc_sh.at[pl.ds(base, RPT)], out1.at[pl.ds(base, RPT)])


@functools.partial(
    pl.kernel,
    out_type=(
        jax.ShapeDtypeStruct((NPAD, 16), jnp.float32),
        jax.ShapeDtypeStruct((NPAD, 16), jnp.float32),
    ),
    mesh=_mesh,
    scratch_types=[
        pltpu.VMEM((CPW, CHUNK), jnp.int32),
        pltpu.VMEM((CHUNK, 16), jnp.float32),
        pltpu.VMEM((RPT, 16), jnp.float32),
        pltpu.VMEM_SHARED((NPAD, 16), jnp.float32),
        pltpu.SemaphoreType.DMA,
    ],
)
def _sc_deg(dst_hbm, deg0, deg1, dstbuf, ones_v, zrows, dacc_sh, sem):
    c = lax.axis_index("c")
    s = lax.axis_index("s")
    wid = s * NC + c

    def _fill1(i, carry):
        ones_v[i, :] = jnp.ones((16,), jnp.float32)
        return carry
    lax.fori_loop(0, CHUNK, _fill1, 0)

    def _fill0(i, carry):
        zrows[i, :] = jnp.zeros((16,), jnp.float32)
        return carry
    lax.fori_loop(0, RPT, _fill0, 0)

    base = s * RPT
    pltpu.sync_copy(zrows, dacc_sh.at[pl.ds(base, RPT)])
    pltpu.sync_copy(dst_hbm.at[wid], dstbuf)
    plsc.subcore_barrier()

    def _body(j, carry):
        pltpu.sync_copy(ones_v, dacc_sh.at[dstbuf.at[j]], add=True)
        return carry
    lax.fori_loop(0, CPW, _body, 0)
    plsc.subcore_barrier()

    @pl.when(c == 0)
    def _():
        pltpu.sync_copy(dacc_sh.at[pl.ds(base, RPT)], deg0.at[pl.ds(base, RPT)])

    @pl.when(c == 1)
    def _():
        pltpu.sync_copy(dacc_sh.at[pl.ds(base, RPT)], deg1.at[pl.ds(base, RPT)])


BM = 256


def _dinv(da_ref, db_ref):
    deg = da_ref[:, 0:1] + db_ref[:, 0:1] + 1.0
    return lax.rsqrt(deg)


def _mm1_body(x_ref, w_ref, da_ref, db_ref, g_ref):
    dinv = _dinv(da_ref, db_ref)
    h = jnp.dot(x_ref[...], w_ref[...], preferred_element_type=jnp.float32)
    g_ref[...] = h * dinv


def _mm2_body(a0_ref, a1_ref, g_ref, da_ref, db_ref, w_ref, b_ref, o_ref):
    dinv = _dinv(da_ref, db_ref)
    y = dinv * (a0_ref[...] + a1_ref[...] + g_ref[...]) + b_ref[...]
    y = jnp.maximum(y, 0.0)
    h = jnp.dot(y, w_ref[...], preferred_element_type=jnp.float32)
    o_ref[...] = h * dinv


def _mm3_body(a0_ref, a1_ref, g_ref, da_ref, db_ref, b_ref, o_ref):
    dinv = _dinv(da_ref, db_ref)
    y = dinv * (a0_ref[...] + a1_ref[...] + g_ref[...]) + b_ref[...]
    o_ref[...] = jnp.maximum(y, 0.0)


def _row_spec(i):
    return (i, 0)


def _rep_spec(i):
    return (0, 0)


_F32 = jnp.float32


def _mm1(x_pad, W1, degA, degB):
    return pl.pallas_call(
        _mm1_body,
        grid=(NPAD // BM,),
        in_specs=[
            pl.BlockSpec((BM, D), _row_spec),
            pl.BlockSpec((D, D), _rep_spec),
            pl.BlockSpec((BM, 16), _row_spec),
            pl.BlockSpec((BM, 16), _row_spec),
        ],
        out_specs=pl.BlockSpec((BM, D), _row_spec),
        out_shape=jax.ShapeDtypeStruct((NPAD, D), _F32),
    )(x_pad, W1, degA, degB)


def _mm2(a0, a1, g1, degA, degB, W2, b1):
    return pl.pallas_call(
        _mm2_body,
        grid=(NPAD // BM,),
        in_specs=[
            pl.BlockSpec((BM, D), _row_spec),
            pl.BlockSpec((BM, D), _row_spec),
            pl.BlockSpec((BM, D), _row_spec),
            pl.BlockSpec((BM, 16), _row_spec),
            pl.BlockSpec((BM, 16), _row_spec),
            pl.BlockSpec((D, D), _rep_spec),
            pl.BlockSpec((1, D), _rep_spec),
        ],
        out_specs=pl.BlockSpec((BM, D), _row_spec),
        out_shape=jax.ShapeDtypeStruct((NPAD, D), _F32),
    )(a0, a1, g1, degA, degB, W2, b1)


def _mm3(a0, a1, g2, degA, degB, b2):
    return pl.pallas_call(
        _mm3_body,
        grid=(NPAD // BM,),
        in_specs=[
            pl.BlockSpec((BM, D), _row_spec),
            pl.BlockSpec((BM, D), _row_spec),
            pl.BlockSpec((BM, D), _row_spec),
            pl.BlockSpec((BM, 16), _row_spec),
            pl.BlockSpec((BM, 16), _row_spec),
            pl.BlockSpec((1, D), _rep_spec),
        ],
        out_specs=pl.BlockSpec((BM, D), _row_spec),
        out_shape=jax.ShapeDtypeStruct((NPAD, D), _F32),
    )(a0, a1, g2, degA, degB, b2)


def kernel(x, edge_index, W1, b1, W2, b2):
    src = edge_index[0].astype(jnp.int32)
    dst = edge_index[1].astype(jnp.int32)
    pad = EPAD - E
    padv = jnp.full((pad,), N, jnp.int32)
    src_p = jnp.concatenate([src, padv]).reshape(NW, CPW, CHUNK)
    dst_p = jnp.concatenate([dst, padv]).reshape(NW, CPW, CHUNK)
    x_pad = jnp.pad(x, ((0, NPAD - N), (0, 0)))
    b1r = b1.reshape(1, D)
    b2r = b2.reshape(1, D)

    degA, degB = _sc_deg(dst_p)
    g1 = _mm1(x_pad, W1, degA, degB)
    a0, a1 = _sc_agg(g1, src_p, dst_p)
    g2 = _mm2(a0, a1, g1, degA, degB, W2, b1r)
    c0, c1 = _sc_agg(g2, src_p, dst_p)
    out = _mm3(c0, c1, g2, degA, degB, b2r)
    return out[:N]


# speedup vs baseline: 4.2492x; 4.2492x over previous
import functools

import jax
import jax.numpy as jnp
from jax import lax
from jax.experimental import pallas as pl
from jax.experimental.pallas import tpu as pltpu
from jax.experimental.pallas import tpu_sc as plsc

N = 10000
D = 128
E = 320000

NPAD = 10240
HALF = NPAD // 2
NC, NS = 2, 16
NW = NC * NS
CHUNK = 128
CPW = 80
EPW = CPW * CHUNK
EPAD = NW * EPW
GROUP = 4
NGROUP = CPW // GROUP
NRING = 2 * GROUP
ACC_ROWS = HALF + 8
RPT = HALF // NS

_mesh = plsc.VectorSubcoreMesh(
    core_axis_name="c", subcore_axis_name="s", num_cores=NC)

_F32 = jnp.float32


@functools.partial(
    pl.kernel,
    out_type=(
        jax.ShapeDtypeStruct((HALF, D), _F32),
        jax.ShapeDtypeStruct((HALF, D), _F32),
    ),
    mesh=_mesh,
    scratch_types=[
        pltpu.VMEM((CPW, CHUNK), jnp.int32),
        pltpu.VMEM((CPW, CHUNK), jnp.int32),
        pltpu.VMEM((GROUP * CHUNK, D), _F32),
        pltpu.VMEM_SHARED((ACC_ROWS, D), _F32),
        pltpu.SemaphoreType.DMA,
        pltpu.SemaphoreType.DMA,
    ],
)
def _sc_agg(g_hbm, src_hbm, dst_hbm, out0, out1,
            srcbuf, dstbuf, rows, acc_sh, gsem, ssem):
    c = lax.axis_index("c")
    s = lax.axis_index("s")
    wid = s * NC + c
    base = s * RPT

    def _zrow(i, carry):
        for j in range(D // 16):
            rows[i, pl.ds(j * 16, 16)] = jnp.zeros((16,), _F32)
        return carry
    lax.fori_loop(0, GROUP * CHUNK, _zrow, 0)

    pltpu.sync_copy(rows.at[pl.ds(0, RPT)], acc_sh.at[pl.ds(base, RPT)])

    @pl.when(s == NS - 1)
    def _():
        pltpu.sync_copy(rows.at[pl.ds(0, ACC_ROWS - HALF)],
                        acc_sh.at[pl.ds(HALF, ACC_ROWS - HALF)])

    pltpu.sync_copy(src_hbm.at[wid], srcbuf)
    pltpu.sync_copy(dst_hbm.at[wid], dstbuf)
    plsc.subcore_barrier()

    def _group(gi, carry):
        gathers = []
        for b in range(GROUP):
            gathers.append(pltpu.async_copy(
                g_hbm.at[srcbuf.at[gi * GROUP + b]],
                rows.at[pl.ds(b * CHUNK, CHUNK)], gsem))
        scatters = []
        for b in range(GROUP):
            gathers[b].wait()
            scatters.append(pltpu.async_copy(
                rows.at[pl.ds(b * CHUNK, CHUNK)],
                acc_sh.at[dstbuf.at[gi * GROUP + b]], ssem, add=True))
        for b in range(GROUP):
            scatters[b].wait()
        return carry
    lax.fori_loop(0, NGROUP, _group, 0)
    plsc.subcore_barrier()

    pltpu.sync_copy(acc_sh.at[pl.ds(base, RPT)], rows.at[pl.ds(0, RPT)])

    @pl.when(c == 0)
    def _():
        pltpu.sync_copy(rows.at[pl.ds(0, RPT)], out0.at[pl.ds(base, RPT)])

    @pl.when(c == 1)
    def _():
        pltpu.sync_copy(rows.at[pl.ds(0, RPT)], out1.at[pl.ds(base, RPT)])


@functools.partial(
    pl.kernel,
    out_type=(
        jax.ShapeDtypeStruct((HALF, D), _F32),
        jax.ShapeDtypeStruct((HALF, D), _F32),
    ),
    mesh=_mesh,
    scratch_types=[
        pltpu.VMEM((CPW, CHUNK), jnp.int32),
        pltpu.VMEM((CHUNK, D), _F32),
        pltpu.VMEM((RPT + 8, D), _F32),
        pltpu.VMEM_SHARED((ACC_ROWS, D), _F32),
        pltpu.SemaphoreType.DMA,
    ],
)
def _sc_deg(dst_hbm, deg0, deg1, dstbuf, ones_v, zrows, dacc_sh, sem):
    c = lax.axis_index("c")
    s = lax.axis_index("s")
    wid = s * NC + c
    base = s * RPT

    def _fill1(i, carry):
        for j in range(D // 16):
            ones_v[i, pl.ds(j * 16, 16)] = jnp.ones((16,), _F32)
        return carry
    lax.fori_loop(0, CHUNK, _fill1, 0)

    def _fill0(i, carry):
        for j in range(D // 16):
            zrows[i, pl.ds(j * 16, 16)] = jnp.zeros((16,), _F32)
        return carry
    lax.fori_loop(0, RPT + 8, _fill0, 0)

    pltpu.sync_copy(dst_hbm.at[wid], dstbuf)
    pltpu.sync_copy(zrows.at[pl.ds(0, RPT)], dacc_sh.at[pl.ds(base, RPT)])

    @pl.when(s == NS - 1)
    def _():
        pltpu.sync_copy(zrows.at[pl.ds(0, ACC_ROWS - HALF)],
                        dacc_sh.at[pl.ds(HALF, ACC_ROWS - HALF)])
    plsc.subcore_barrier()

    def _body(j, carry):
        pltpu.sync_copy(ones_v, dacc_sh.at[dstbuf.at[j]], add=True)
        return carry
    lax.fori_loop(0, CPW, _body, 0)
    plsc.subcore_barrier()

    pltpu.sync_copy(dacc_sh.at[pl.ds(base, RPT)], zrows.at[pl.ds(0, RPT)])

    @pl.when(c == 0)
    def _():
        pltpu.sync_copy(zrows.at[pl.ds(0, RPT)], deg0.at[pl.ds(base, RPT)])

    @pl.when(c == 1)
    def _():
        pltpu.sync_copy(zrows.at[pl.ds(0, RPT)], deg1.at[pl.ds(base, RPT)])


BM = 256


def _dinv(da_ref, db_ref):
    deg = da_ref[:, 0:1] + db_ref[:, 0:1] + 1.0
    return lax.rsqrt(deg)


def _mm1_body(x_ref, w_ref, da_ref, db_ref, g_ref):
    dinv = _dinv(da_ref, db_ref)
    h = jnp.dot(x_ref[...], w_ref[...], preferred_element_type=_F32)
    g_ref[...] = h * dinv


def _mm2_body(a0_ref, a1_ref, g_ref, da_ref, db_ref, b_ref, o_ref):
    dinv = _dinv(da_ref, db_ref)
    y = dinv * (a0_ref[...] + a1_ref[...] + g_ref[...]) + b_ref[...]
    o_ref[...] = jnp.maximum(y, 0.0)


def _row_spec(i):
    return (i, 0)


def _rep_spec(i):
    return (0, 0)


def _mm1(x_pad, W, degA, degB):
    return pl.pallas_call(
        _mm1_body,
        grid=(NPAD // BM,),
        in_specs=[
            pl.BlockSpec((BM, D), _row_spec),
            pl.BlockSpec((D, D), _rep_spec),
            pl.BlockSpec((BM, D), _row_spec),
            pl.BlockSpec((BM, D), _row_spec),
        ],
        out_specs=pl.BlockSpec((BM, D), _row_spec),
        out_shape=jax.ShapeDtypeStruct((NPAD, D), _F32),
    )(x_pad, W, degA, degB)


def _mm2(a0, a1, g, degA, degB, b):
    return pl.pallas_call(
        _mm2_body,
        grid=(NPAD // BM,),
        in_specs=[
            pl.BlockSpec((BM, D), _row_spec),
            pl.BlockSpec((BM, D), _row_spec),
            pl.BlockSpec((BM, D), _row_spec),
            pl.BlockSpec((BM, D), _row_spec),
            pl.BlockSpec((BM, D), _row_spec),
            pl.BlockSpec((1, D), _rep_spec),
        ],
        out_specs=pl.BlockSpec((BM, D), _row_spec),
        out_shape=jax.ShapeDtypeStruct((NPAD, D), _F32),
    )(a0, a1, g, degA, degB, b)


def kernel(x, edge_index, W1, b1, W2, b2):
    src = edge_index[0].astype(jnp.int32)
    dst = edge_index[1].astype(jnp.int32)
    pad = EPAD - E
    srcf = jnp.concatenate([src, jnp.full((pad,), N, jnp.int32)])
    dstf = jnp.concatenate([dst, jnp.full((pad,), N, jnp.int32)])
    src_p = srcf.reshape(NW, CPW, CHUNK)
    dst_p = dstf.reshape(NW, CPW, CHUNK)
    dst_h0 = jnp.minimum(dstf, HALF).reshape(NW, CPW, CHUNK)
    dst_h1 = jnp.where(dstf >= HALF, dstf - HALF, HALF).reshape(NW, CPW, CHUNK)
    x_pad = jnp.pad(x, ((0, NPAD - N), (0, 0)))
    W_stack = jnp.stack([W1, W2])
    b_stack = jnp.stack([b1.reshape(1, D), b2.reshape(1, D)])

    d00, d10 = _sc_deg(dst_h0)
    d01, d11 = _sc_deg(dst_h1)
    degA = jnp.concatenate([d00, d01])
    degB = jnp.concatenate([d10, d11])

    def _layer(inp, Wb):
        W, b = Wb
        g = _mm1(inp, W, degA, degB)
        p00, p10 = _sc_agg(g, src_p, dst_h0)
        p01, p11 = _sc_agg(g, src_p, dst_h1)
        a0 = jnp.concatenate([p00, p01])
        a1 = jnp.concatenate([p10, p11])
        y = _mm2(a0, a1, g, degA, degB, b)
        return y, None

    out, _ = lax.scan(_layer, x_pad, (W_stack, b_stack))
    return out[:N]


# speedup vs baseline: 4.4420x; 1.0454x over previous
import functools

import jax
import jax.numpy as jnp
from jax import lax
from jax.experimental import pallas as pl
from jax.experimental.pallas import tpu as pltpu
from jax.experimental.pallas import tpu_sc as plsc

N = 10000
D = 128
E = 320000

NPAD = 10240
HALF = NPAD // 2
NC, NS = 2, 16
NW = NC * NS
CHUNK = 128
CPW = 80
EPW = CPW * CHUNK
EPAD = NW * EPW
GROUP = 4
NGROUP = CPW // GROUP
NDUMP = 128
ACC_ROWS = HALF + NDUMP
RPT = HALF // NS

_mesh = plsc.VectorSubcoreMesh(
    core_axis_name="c", subcore_axis_name="s", num_cores=NC)

_F32 = jnp.float32


@functools.partial(
    pl.kernel,
    out_type=(
        jax.ShapeDtypeStruct((HALF, D), _F32),
        jax.ShapeDtypeStruct((HALF, D), _F32),
    ),
    mesh=_mesh,
    scratch_types=[
        pltpu.VMEM((CPW, CHUNK), jnp.int32),
        pltpu.VMEM((CPW, CHUNK), jnp.int32),
        pltpu.VMEM((GROUP * CHUNK, D), _F32),
        pltpu.VMEM_SHARED((ACC_ROWS, D), _F32),
        pltpu.SemaphoreType.DMA,
        pltpu.SemaphoreType.DMA,
    ],
)
def _sc_agg(g_hbm, src_hbm, dst_hbm, out0, out1,
            srcbuf, dstbuf, rows, acc_sh, gsem, ssem):
    c = lax.axis_index("c")
    s = lax.axis_index("s")
    wid = s * NC + c
    base = s * RPT

    def _zrow(i, carry):
        for j in range(D // 16):
            rows[i, pl.ds(j * 16, 16)] = jnp.zeros((16,), _F32)
        return carry
    lax.fori_loop(0, GROUP * CHUNK, _zrow, 0)

    pltpu.sync_copy(rows.at[pl.ds(0, RPT)], acc_sh.at[pl.ds(base, RPT)])

    @pl.when(s == NS - 1)
    def _():
        pltpu.sync_copy(rows.at[pl.ds(0, ACC_ROWS - HALF)],
                        acc_sh.at[pl.ds(HALF, ACC_ROWS - HALF)])

    pltpu.sync_copy(src_hbm.at[wid], srcbuf)
    pltpu.sync_copy(dst_hbm.at[wid], dstbuf)
    plsc.subcore_barrier()

    def _group(gi, carry):
        gathers = []
        for b in range(GROUP):
            gathers.append(pltpu.async_copy(
                g_hbm.at[srcbuf.at[gi * GROUP + b]],
                rows.at[pl.ds(b * CHUNK, CHUNK)], gsem))
        scatters = []
        for b in range(GROUP):
            gathers[b].wait()
            scatters.append(pltpu.async_copy(
                rows.at[pl.ds(b * CHUNK, CHUNK)],
                acc_sh.at[dstbuf.at[gi * GROUP + b]], ssem, add=True))
        for b in range(GROUP):
            scatters[b].wait()
        return carry
    lax.fori_loop(0, NGROUP, _group, 0)
    plsc.subcore_barrier()

    pltpu.sync_copy(acc_sh.at[pl.ds(base, RPT)], rows.at[pl.ds(0, RPT)])

    @pl.when(c == 0)
    def _():
        pltpu.sync_copy(rows.at[pl.ds(0, RPT)], out0.at[pl.ds(base, RPT)])

    @pl.when(c == 1)
    def _():
        pltpu.sync_copy(rows.at[pl.ds(0, RPT)], out1.at[pl.ds(base, RPT)])


@functools.partial(
    pl.kernel,
    out_type=(
        jax.ShapeDtypeStruct((HALF, D), _F32),
        jax.ShapeDtypeStruct((HALF, D), _F32),
    ),
    mesh=_mesh,
    scratch_types=[
        pltpu.VMEM((CPW, CHUNK), jnp.int32),
        pltpu.VMEM((CHUNK, D), _F32),
        pltpu.VMEM((RPT, D), _F32),
        pltpu.VMEM_SHARED((ACC_ROWS, D), _F32),
        pltpu.SemaphoreType.DMA,
    ],
)
def _sc_deg(dst_hbm, deg0, deg1, dstbuf, ones_v, zrows, dacc_sh, sem):
    c = lax.axis_index("c")
    s = lax.axis_index("s")
    wid = s * NC + c
    base = s * RPT

    def _fill1(i, carry):
        for j in range(D // 16):
            ones_v[i, pl.ds(j * 16, 16)] = jnp.ones((16,), _F32)
        return carry
    lax.fori_loop(0, CHUNK, _fill1, 0)

    def _fill0(i, carry):
        for j in range(D // 16):
            zrows[i, pl.ds(j * 16, 16)] = jnp.zeros((16,), _F32)
        return carry
    lax.fori_loop(0, RPT, _fill0, 0)

    pltpu.sync_copy(dst_hbm.at[wid], dstbuf)
    pltpu.sync_copy(zrows.at[pl.ds(0, RPT)], dacc_sh.at[pl.ds(base, RPT)])

    @pl.when(s == NS - 1)
    def _():
        pltpu.sync_copy(zrows.at[pl.ds(0, ACC_ROWS - HALF)],
                        dacc_sh.at[pl.ds(HALF, ACC_ROWS - HALF)])
    plsc.subcore_barrier()

    def _body(j, carry):
        pltpu.sync_copy(ones_v, dacc_sh.at[dstbuf.at[j]], add=True)
        return carry
    lax.fori_loop(0, CPW, _body, 0)
    plsc.subcore_barrier()

    pltpu.sync_copy(dacc_sh.at[pl.ds(base, RPT)], zrows.at[pl.ds(0, RPT)])

    @pl.when(c == 0)
    def _():
        pltpu.sync_copy(zrows.at[pl.ds(0, RPT)], deg0.at[pl.ds(base, RPT)])

    @pl.when(c == 1)
    def _():
        pltpu.sync_copy(zrows.at[pl.ds(0, RPT)], deg1.at[pl.ds(base, RPT)])


BM = 256


def _dinv(da_ref, db_ref):
    deg = da_ref[:, 0:1] + db_ref[:, 0:1] + 1.0
    return lax.rsqrt(deg)


def _mm1_body(x_ref, w_ref, da_ref, db_ref, g_ref):
    dinv = _dinv(da_ref, db_ref)
    h = jnp.dot(x_ref[...], w_ref[...], preferred_element_type=_F32)
    g_ref[...] = h * dinv


def _mm2_body(a0_ref, a1_ref, g_ref, da_ref, db_ref, b_ref, o_ref):
    dinv = _dinv(da_ref, db_ref)
    y = dinv * (a0_ref[...] + a1_ref[...] + g_ref[...]) + b_ref[...]
    o_ref[...] = jnp.maximum(y, 0.0)


def _row_spec(i):
    return (i, 0)


def _rep_spec(i):
    return (0, 0)


def _mm1(x_pad, W, degA, degB):
    return pl.pallas_call(
        _mm1_body,
        grid=(NPAD // BM,),
        in_specs=[
            pl.BlockSpec((BM, D), _row_spec),
            pl.BlockSpec((D, D), _rep_spec),
            pl.BlockSpec((BM, D), _row_spec),
            pl.BlockSpec((BM, D), _row_spec),
        ],
        out_specs=pl.BlockSpec((BM, D), _row_spec),
        out_shape=jax.ShapeDtypeStruct((NPAD, D), _F32),
    )(x_pad, W, degA, degB)


def _mm2(a0, a1, g, degA, degB, b):
    return pl.pallas_call(
        _mm2_body,
        grid=(NPAD // BM,),
        in_specs=[
            pl.BlockSpec((BM, D), _row_spec),
            pl.BlockSpec((BM, D), _row_spec),
            pl.BlockSpec((BM, D), _row_spec),
            pl.BlockSpec((BM, D), _row_spec),
            pl.BlockSpec((BM, D), _row_spec),
            pl.BlockSpec((1, D), _rep_spec),
        ],
        out_specs=pl.BlockSpec((BM, D), _row_spec),
        out_shape=jax.ShapeDtypeStruct((NPAD, D), _F32),
    )(a0, a1, g, degA, degB, b)


def kernel(x, edge_index, W1, b1, W2, b2):
    src = edge_index[0].astype(jnp.int32)
    dst = edge_index[1].astype(jnp.int32)
    pad = EPAD - E
    srcf = jnp.concatenate([src, jnp.full((pad,), N, jnp.int32)])
    dstf = jnp.concatenate([dst, jnp.full((pad,), N, jnp.int32)])
    src_p = srcf.reshape(NW, CPW, CHUNK)
    dst_p = dstf.reshape(NW, CPW, CHUNK)
    dump = HALF + (jnp.arange(EPAD, dtype=jnp.int32) % NDUMP)
    dst_h0 = jnp.where(dstf < HALF, dstf, dump).reshape(NW, CPW, CHUNK)
    dst_h1 = jnp.where(dstf >= HALF, dstf - HALF, dump).reshape(NW, CPW, CHUNK)
    x_pad = jnp.pad(x, ((0, NPAD - N), (0, 0)))
    W_stack = jnp.stack([W1, W2])
    b_stack = jnp.stack([b1.reshape(1, D), b2.reshape(1, D)])

    d00, d10 = _sc_deg(dst_h0)
    d01, d11 = _sc_deg(dst_h1)
    degA = jnp.concatenate([d00, d01])
    degB = jnp.concatenate([d10, d11])

    def _layer(inp, Wb):
        W, b = Wb
        g = _mm1(inp, W, degA, degB)
        p00, p10 = _sc_agg(g, src_p, dst_h0)
        p01, p11 = _sc_agg(g, src_p, dst_h1)
        a0 = jnp.concatenate([p00, p01])
        a1 = jnp.concatenate([p10, p11])
        y = _mm2(a0, a1, g, degA, degB, b)
        return y, None

    out, _ = lax.scan(_layer, x_pad, (W_stack, b_stack))
    return out[:N]
